# baseline (speedup 1.0000x reference)
import functools

import jax
import jax.numpy as jnp
from jax import lax
from jax.experimental import pallas as pl
from jax.experimental.pallas import tpu as pltpu
from jax.experimental.pallas import tpu_sc as plsc

_VOCAB = 1000000
_D = 64
_R = 16
_SCALE = 1.0 / _R
_NC = 2
_NS = 16
_NW = _NC * _NS
_CHUNK = 128
_G = 4
_W = 128


def _sc_embed_lora(idx, wcomb, btt, fields, batch):
    n_total = fields * batch
    n_per_w = n_total // _NW
    ch_per_w = n_per_w // _CHUNK
    ch_per_f = batch // _CHUNK
    mesh = plsc.VectorSubcoreMesh(core_axis_name="c", subcore_axis_name="s")

    @functools.partial(
        pl.kernel,
        out_type=jax.ShapeDtypeStruct((fields, _D // 8, batch // _CHUNK,
                                       8, _CHUNK), jnp.float32),
        mesh=mesh,
        scratch_types=[
            pltpu.VMEM((n_per_w,), jnp.int32),
            pltpu.VMEM((2, _CHUNK, _W), jnp.float32),
            pltpu.VMEM((2, _D // 8, 8, _CHUNK), jnp.float32),
            pltpu.VMEM((_R, _D), jnp.float32),
            pltpu.SemaphoreType.DMA,
            pltpu.SemaphoreType.DMA,
            pltpu.SemaphoreType.DMA,
            pltpu.SemaphoreType.DMA,
        ],
        compiler_params=pltpu.CompilerParams(
            use_tc_tiling_on_sc=False, needs_layout_passes=False),
    )
    def k(idx_hbm, w_hbm, btt_hbm, out_hbm,
          idx_v, base2_v, tr2_v, btt_v, sem_w0, sem_w1, sem_o0, sem_o1):
        wid = lax.axis_index("s") * _NC + lax.axis_index("c")
        c0 = wid * ch_per_w
        pltpu.sync_copy(idx_hbm.at[pl.ds(wid * n_per_w, n_per_w)], idx_v)
        pltpu.sync_copy(btt_hbm, btt_v)
        iota16 = lax.iota(jnp.int32, 16)
        jb_rows = [(iota16 + 16 * c) // 8 for c in range(4)]
        jr_rows = [(iota16 + 16 * c) % 8 for c in range(4)]
        sems_w = (sem_w0, sem_w1)
        sems_o = (sem_o0, sem_o1)

        def fire_gather(g, p):
            off = g * _CHUNK
            idx_sl = idx_v.at[pl.ds(off, _CHUNK)]
            pltpu.async_copy(w_hbm.at[idx_sl], base2_v.at[p], sems_w[p])

        def out_ref(g):
            gc = c0 + g
            return out_hbm.at[gc // ch_per_f, :, gc % ch_per_f]

        fire_gather(0, 0)

        def pair_body(g2, _):
            for p in range(2):
                g = g2 * 2 + p
                base_v = base2_v.at[p]
                tr_v = tr2_v.at[p]
                pltpu.make_async_copy(
                    w_hbm.at[idx_v.at[pl.ds(0, _CHUNK)]],
                    base_v, sems_w[p]).wait()

                @pl.when(g + 1 < ch_per_w)
                def _():
                    fire_gather(g + 1, 1 - p)

                @pl.when(g2 >= 1)
                def _():
                    pltpu.make_async_copy(tr_v, out_ref(g), sems_o[p]).wait()

                def gbody(b, _):
                    i0 = b * _G
                    avs = [base_v[i0 + i, pl.ds(_D, _R)] for i in range(_G)]
                    accs = [[base_v[i0 + i, pl.ds(16 * c, 16)]
                             for c in range(4)] for i in range(_G)]
                    for kb in range(0, _R, 4):
                        btk = [[btt_v[kb + t, pl.ds(16 * c, 16)]
                                for c in range(4)] for t in range(4)]
                        for i in range(_G):
                            for t in range(4):
                                ak = avs[i][kb + t]
                                for c in range(4):
                                    accs[i][c] = accs[i][c] + ak * btk[t][c]
                    for i in range(_G):
                        coli = jnp.full((16,), i0 + i, jnp.int32)
                        for c in range(4):
                            plsc.store_scatter(
                                tr_v, [jb_rows[c], jr_rows[c], coli],
                                accs[i][c])
                    return 0

                lax.fori_loop(0, _CHUNK // _G, gbody, 0)
                pltpu.async_copy(tr_v, out_ref(g), sems_o[p])
            return 0

        lax.fori_loop(0, ch_per_w // 2, pair_body, 0)
        pltpu.make_async_copy(tr2_v.at[0], out_ref(ch_per_w - 2),
                              sems_o[0]).wait()
        pltpu.make_async_copy(tr2_v.at[1], out_ref(ch_per_w - 1),
                              sems_o[1]).wait()

    return k(idx, wcomb, btt)


def kernel(input_, weight, lora_left_weight, lora_right_weight):
    b, f = input_.shape
    idx = input_.T.reshape(-1).astype(jnp.int32)
    wcomb = jnp.concatenate(
        [weight, lora_left_weight.T,
         jnp.zeros((weight.shape[0], _W - _D - _R), jnp.float32)], axis=1)
    btt = lora_right_weight.T * jnp.float32(_SCALE)
    out5 = _sc_embed_lora(idx, wcomb, btt, f, b)
    out = out5.transpose(2, 4, 0, 1, 3).reshape(b, f, _D)
    return out


# speedup vs baseline: 1.1843x; 1.1843x over previous
import functools

import jax
import jax.numpy as jnp
from jax import lax
from jax.experimental import pallas as pl
from jax.experimental.pallas import tpu as pltpu
from jax.experimental.pallas import tpu_sc as plsc

_VOCAB = 1000000
_D = 64
_R = 16
_SCALE = 1.0 / _R
_NC = 2
_NS = 16
_NW = _NC * _NS
_CHUNK = 128
_G = 8
_W = 128


def _sc_embed_lora(idx, wcomb, btt, fields, batch):
    n_total = fields * batch
    n_per_w = n_total // _NW
    ch_per_w = n_per_w // _CHUNK
    ch_per_f = batch // _CHUNK
    mesh = plsc.VectorSubcoreMesh(core_axis_name="c", subcore_axis_name="s")

    @functools.partial(
        pl.kernel,
        out_type=jax.ShapeDtypeStruct((fields, _D // 8, batch // _CHUNK,
                                       8, _CHUNK), jnp.float32),
        mesh=mesh,
        scratch_types=[
            pltpu.VMEM((n_per_w,), jnp.int32),
            pltpu.VMEM((2, _CHUNK, _W), jnp.float32),
            pltpu.VMEM((2, _D // 8, 8, _CHUNK), jnp.float32),
            pltpu.VMEM((_R, _D), jnp.float32),
            pltpu.SemaphoreType.DMA,
            pltpu.SemaphoreType.DMA,
            pltpu.SemaphoreType.DMA,
            pltpu.SemaphoreType.DMA,
        ],
        compiler_params=pltpu.CompilerParams(
            use_tc_tiling_on_sc=False, needs_layout_passes=False),
    )
    def k(idx_hbm, w_hbm, btt_hbm, out_hbm,
          idx_v, base2_v, tr2_v, btt_v, sem_w0, sem_w1, sem_o0, sem_o1):
        wid = lax.axis_index("s") * _NC + lax.axis_index("c")
        c0 = wid * ch_per_w
        pltpu.sync_copy(idx_hbm.at[pl.ds(wid * n_per_w, n_per_w)], idx_v)
        pltpu.sync_copy(btt_hbm, btt_v)
        iota16 = lax.iota(jnp.int32, 16)
        jb_rows = [(iota16 + 16 * c) // 8 for c in range(4)]
        jr_rows = [(iota16 + 16 * c) % 8 for c in range(4)]
        sems_w = (sem_w0, sem_w1)
        sems_o = (sem_o0, sem_o1)

        def fire_gather(g, p):
            off = g * _CHUNK
            idx_sl = idx_v.at[pl.ds(off, _CHUNK)]
            pltpu.async_copy(w_hbm.at[idx_sl], base2_v.at[p], sems_w[p])

        def out_ref(g):
            gc = c0 + g
            return out_hbm.at[gc // ch_per_f, :, gc % ch_per_f]

        fire_gather(0, 0)

        def pair_body(g2, _):
            for p in range(2):
                g = g2 * 2 + p
                base_v = base2_v.at[p]
                tr_v = tr2_v.at[p]
                pltpu.make_async_copy(
                    w_hbm.at[idx_v.at[pl.ds(0, _CHUNK)]],
                    base_v, sems_w[p]).wait()

                @pl.when(g + 1 < ch_per_w)
                def _():
                    fire_gather(g + 1, 1 - p)

                @pl.when(g2 >= 1)
                def _():
                    pltpu.make_async_copy(tr_v, out_ref(g), sems_o[p]).wait()

                def gbody(b, _):
                    i0 = b * _G
                    avs = [base_v[i0 + i, pl.ds(_D, _R)] for i in range(_G)]
                    accs = [[base_v[i0 + i, pl.ds(16 * c, 16)]
                             for c in range(4)] for i in range(_G)]
                    for kb in range(0, _R, 4):
                        btk = [[btt_v[kb + t, pl.ds(16 * c, 16)]
                                for c in range(4)] for t in range(4)]
                        for i in range(_G):
                            for t in range(4):
                                ak = avs[i][kb + t]
                                for c in range(4):
                                    accs[i][c] = accs[i][c] + ak * btk[t][c]
                    for i in range(_G):
                        coli = jnp.full((16,), i0 + i, jnp.int32)
                        for c in range(4):
                            plsc.store_scatter(
                                tr_v, [jb_rows[c], jr_rows[c], coli],
                                accs[i][c])
                    return 0

                lax.fori_loop(0, _CHUNK // _G, gbody, 0)
                pltpu.async_copy(tr_v, out_ref(g), sems_o[p])
            return 0

        lax.fori_loop(0, ch_per_w // 2, pair_body, 0)
        pltpu.make_async_copy(tr2_v.at[0], out_ref(ch_per_w - 2),
                              sems_o[0]).wait()
        pltpu.make_async_copy(tr2_v.at[1], out_ref(ch_per_w - 1),
                              sems_o[1]).wait()

    return k(idx, wcomb, btt)


def kernel(input_, weight, lora_left_weight, lora_right_weight):
    b, f = input_.shape
    idx = input_.T.reshape(-1).astype(jnp.int32)
    p1 = jnp.pad(jnp.eye(_D, dtype=jnp.float32), ((0, 0), (0, _W - _D)))
    p2 = jnp.pad(jnp.eye(_R, dtype=jnp.float32),
                 ((0, 0), (_D, _W - _D - _R)))
    wcomb = jax.lax.dot_general(
        weight, p1, (((1,), (0,)), ((), ()))) + jax.lax.dot_general(
        lora_left_weight, p2, (((0,), (0,)), ((), ())))
    btt = lora_right_weight.T * jnp.float32(_SCALE)
    out5 = _sc_embed_lora(idx, wcomb, btt, f, b)
    out = out5.transpose(2, 4, 0, 1, 3).reshape(b, f, _D)
    return out
